# Initial kernel scaffold; baseline (speedup 1.0000x reference)
import jax
import jax.numpy as jnp
from jax.experimental import pallas as pl


def kernel(x, edge_index, W1, b1, W2, b2):
    raise NotImplementedError("write your pallas kernel here")


# speedup vs baseline: 44.1885x; 44.1885x over previous
import functools
import math

import jax
import jax.numpy as jnp
from jax import lax
from jax.experimental import pallas as pl
from jax.experimental.pallas import tpu as pltpu
from jax.experimental.pallas import tpu_sc as plsc

NC = 2
NS = 16
NW = NC * NS
LANES = 128
GROUPS = 16
CHUNK = GROUPS * LANES
ZROWS = 800


def _node_pad(n):
  blk = NS * ZROWS
  return ((n + 1 + blk - 1) // blk) * blk


def _hist_kernel(n_sp, rows_per_tile, n_chunks):
  mesh = plsc.VectorSubcoreMesh(core_axis_name="c", subcore_axis_name="s")
  zslice = n_sp // NS

  def body(dst_hbm, out_hbm, idx_v, ones_v, zb_v, hist_sp):
    c = lax.axis_index("c")
    s = lax.axis_index("s")
    wid = s * NC + c

    def fill_z(i, _):
      zb_v[pl.ds(i * 16, 16)] = jnp.zeros((16,), jnp.float32)
      return _
    lax.fori_loop(0, zslice // 16, fill_z, None)

    def fill_o(i, _):
      ones_v[pl.ds(i * 16, 16)] = jnp.ones((16,), jnp.float32)
      return _
    lax.fori_loop(0, LANES // 16, fill_o, None)

    pltpu.sync_copy(zb_v, hist_sp.at[pl.ds(s * zslice, zslice)])
    plsc.subcore_barrier()

    def chunk(i, _):
      row0 = (wid * n_chunks + i) * GROUPS
      pltpu.sync_copy(dst_hbm.at[pl.ds(row0, GROUPS)], idx_v)
      for j in range(GROUPS):
        pltpu.sync_copy(ones_v, hist_sp.at[idx_v.at[j]], add=True)
      return _
    lax.fori_loop(0, n_chunks, chunk, None)

    plsc.subcore_barrier()
    pltpu.sync_copy(hist_sp.at[pl.ds(s * zslice, zslice)],
                    out_hbm.at[c, pl.ds(s * zslice, zslice)])

  return pl.kernel(
      body,
      out_type=jax.ShapeDtypeStruct((NC, n_sp), jnp.float32),
      mesh=mesh,
      compiler_params=pltpu.CompilerParams(use_tc_tiling_on_sc=False),
      scratch_types=[
          pltpu.VMEM((GROUPS, LANES), jnp.int32),
          pltpu.VMEM((LANES,), jnp.float32),
          pltpu.VMEM((zslice,), jnp.float32),
          pltpu.VMEM_SHARED((n_sp,), jnp.float32),
      ],
  )


def _agg_kernel(n_sp, d, n_chunks):
  mesh = plsc.VectorSubcoreMesh(core_axis_name="c", subcore_axis_name="s")
  zslice = n_sp // NS

  def body(src_hbm, dst_hbm, table_hbm, out_hbm,
           src_v, dst_v, rows_v, zb_v, acc_sp, sem0, sem1):
    c = lax.axis_index("c")
    s = lax.axis_index("s")
    wid = s * NC + c

    def fill_z(i, _):
      zb_v[i, :] = jnp.zeros((16,), jnp.float32)
      return _
    lax.fori_loop(0, ZROWS, fill_z, None)
    for k in range(zslice // ZROWS):
      pltpu.sync_copy(zb_v, acc_sp.at[pl.ds(s * zslice + k * ZROWS, ZROWS)])
    plsc.subcore_barrier()

    sems = (sem0, sem1)

    def chunk(i, _):
      row0 = (wid * n_chunks + i) * GROUPS
      pltpu.sync_copy(src_hbm.at[pl.ds(row0, GROUPS)], src_v)
      pltpu.sync_copy(dst_hbm.at[pl.ds(row0, GROUPS)], dst_v)
      cps = [None, None]
      cps[0] = pltpu.async_copy(table_hbm.at[src_v.at[0]], rows_v.at[0], sem0)
      for j in range(GROUPS):
        b = j & 1
        cps[b].wait()
        if j + 1 < GROUPS:
          cps[1 - b] = pltpu.async_copy(
              table_hbm.at[src_v.at[j + 1]], rows_v.at[1 - b], sems[1 - b])
        pltpu.sync_copy(rows_v.at[b], acc_sp.at[dst_v.at[j]], add=True)
      return _
    lax.fori_loop(0, n_chunks, chunk, None)

    plsc.subcore_barrier()
    pltpu.sync_copy(acc_sp.at[pl.ds(s * zslice, zslice)],
                    out_hbm.at[c, pl.ds(s * zslice, zslice)])

  return pl.kernel(
      body,
      out_type=jax.ShapeDtypeStruct((NC, n_sp, d), jnp.float32),
      mesh=mesh,
      compiler_params=pltpu.CompilerParams(use_tc_tiling_on_sc=False),
      scratch_types=[
          pltpu.VMEM((GROUPS, LANES), jnp.int32),
          pltpu.VMEM((GROUPS, LANES), jnp.int32),
          pltpu.VMEM((2, LANES, d), jnp.float32),
          pltpu.VMEM((ZROWS, d), jnp.float32),
          pltpu.VMEM_SHARED((n_sp, d), jnp.float32),
          pltpu.SemaphoreType.DMA,
          pltpu.SemaphoreType.DMA,
      ],
  )


_BR = 2000


def _stage_a_body(dp0, dp1, x, w1, dis_o, g1_o):
  deg = dp0[:] + dp1[:] + 1.0
  dis = lax.rsqrt(deg)
  dis_o[:] = dis
  g1_o[:] = jnp.dot(x[:], w1[:], preferred_element_type=jnp.float32) * dis


def _stage_b_body(a0, a1, g1, dis, b1, u_o):
  total = (a0[:] + a1[:] + g1[:]) * dis[:] + b1[:]
  u_o[:] = jnp.maximum(total, 0.0) * dis[:]


def _stage_c_body(a0, a1, u, dis, w2, b2, out_o):
  v = (a0[:] + a1[:] + u[:]) * dis[:]
  o = jnp.dot(v, w2[:], preferred_element_type=jnp.float32) + b2[:]
  m = jnp.max(o, axis=1, keepdims=True)
  lse = m + jnp.log(jnp.sum(jnp.exp(o - m), axis=1, keepdims=True))
  out_o[:] = o - lse


def _row_spec(d):
  return pl.BlockSpec((_BR, d), lambda i: (i, 0))


def _full_spec(r, d):
  return pl.BlockSpec((r, d), lambda i: (0, 0))


def kernel(x, edge_index, W1, b1, W2, b2):
  n, d_in = x.shape
  e = edge_index.shape[1]
  d_hid = W1.shape[1]
  d_out = W2.shape[1]
  n_sp = _node_pad(n)
  pad_row = n

  n_chunks = math.ceil(e / (NW * CHUNK))
  e_pad = NW * n_chunks * CHUNK
  src = edge_index[0]
  dst = edge_index[1]
  if e_pad > e:
    src = jnp.concatenate([src, jnp.zeros((e_pad - e,), src.dtype)])
    dst = jnp.concatenate([dst, jnp.full((e_pad - e,), pad_row, dst.dtype)])
  src2 = src.reshape(-1, LANES)
  dst2 = dst.reshape(-1, LANES)

  hist = _hist_kernel(n_sp, n_sp // NS, n_chunks)(dst2)
  dp0 = hist[0, :n].reshape(n, 1)
  dp1 = hist[1, :n].reshape(n, 1)

  grid = (n // _BR,)
  dis, g1 = pl.pallas_call(
      _stage_a_body,
      grid=grid,
      in_specs=[_row_spec(1), _row_spec(1), _row_spec(d_in),
                _full_spec(d_in, d_hid)],
      out_specs=[_row_spec(1), _row_spec(d_hid)],
      out_shape=[jax.ShapeDtypeStruct((n, 1), jnp.float32),
                 jax.ShapeDtypeStruct((n, d_hid), jnp.float32)],
  )(dp0, dp1, x, W1)

  agg = _agg_kernel(n_sp, d_hid, n_chunks)
  acc1 = agg(src2, dst2, g1)

  u = pl.pallas_call(
      _stage_b_body,
      grid=grid,
      in_specs=[_row_spec(d_hid), _row_spec(d_hid), _row_spec(d_hid),
                _row_spec(1), _full_spec(1, d_hid)],
      out_specs=_row_spec(d_hid),
      out_shape=jax.ShapeDtypeStruct((n, d_hid), jnp.float32),
  )(acc1[0, :n], acc1[1, :n], g1, dis, b1.reshape(1, d_hid))

  acc2 = agg(src2, dst2, u)

  out = pl.pallas_call(
      _stage_c_body,
      grid=grid,
      in_specs=[_row_spec(d_hid), _row_spec(d_hid), _row_spec(d_hid),
                _row_spec(1), _full_spec(d_hid, d_out),
                _full_spec(1, d_out)],
      out_specs=_row_spec(d_out),
      out_shape=jax.ShapeDtypeStruct((n, d_out), jnp.float32),
  )(acc2[0, :n], acc2[1, :n], u, dis, W2, b2.reshape(1, d_out))

  return out


# speedup vs baseline: 77.0023x; 1.7426x over previous
import functools
import math

import jax
import jax.numpy as jnp
from jax import lax
from jax.experimental import pallas as pl
from jax.experimental.pallas import tpu as pltpu
from jax.experimental.pallas import tpu_sc as plsc

NC = 2
NS = 16
NW = NC * NS
LANES = 128
GROUPS = 32
CHUNK = GROUPS * LANES
ZROWS = 400
NBUF = 6
PRIME = 3


def _node_pad(n):
  blk = NS * ZROWS
  return ((n + 1 + blk - 1) // blk) * blk


def _hist_kernel(n_sp, rows_per_tile, n_chunks):
  mesh = plsc.VectorSubcoreMesh(core_axis_name="c", subcore_axis_name="s")
  zslice = n_sp // NS

  def body(dst_hbm, out_hbm, idx_v, ones_v, zb_v, hist_sp, hsem):
    c = lax.axis_index("c")
    s = lax.axis_index("s")
    wid = s * NC + c

    def fill_z(i, _):
      zb_v[pl.ds(i * 16, 16)] = jnp.zeros((16,), jnp.float32)
      return _
    lax.fori_loop(0, zslice // 16, fill_z, None)

    def fill_o(i, _):
      ones_v[pl.ds(i * 16, 16)] = jnp.ones((16,), jnp.float32)
      return _
    lax.fori_loop(0, LANES // 16, fill_o, None)

    pltpu.sync_copy(zb_v, hist_sp.at[pl.ds(s * zslice, zslice)])
    plsc.subcore_barrier()

    def chunk(i, _):
      row0 = (wid * n_chunks + i) * GROUPS
      pltpu.sync_copy(dst_hbm.at[pl.ds(row0, GROUPS)], idx_v)
      hs = []
      for j in range(GROUPS):
        hs.append(pltpu.async_copy(
            ones_v, hist_sp.at[idx_v.at[j]], hsem, add=True))
      for h in hs:
        h.wait()
      return _
    lax.fori_loop(0, n_chunks, chunk, None)

    plsc.subcore_barrier()
    pltpu.sync_copy(hist_sp.at[pl.ds(s * zslice, zslice)],
                    out_hbm.at[c, pl.ds(s * zslice, zslice)])

  return pl.kernel(
      body,
      out_type=jax.ShapeDtypeStruct((NC, n_sp), jnp.float32),
      mesh=mesh,
      compiler_params=pltpu.CompilerParams(use_tc_tiling_on_sc=False),
      scratch_types=[
          pltpu.VMEM((GROUPS, LANES), jnp.int32),
          pltpu.VMEM((LANES,), jnp.float32),
          pltpu.VMEM((zslice,), jnp.float32),
          pltpu.VMEM_SHARED((n_sp,), jnp.float32),
          pltpu.SemaphoreType.DMA,
      ],
  )


def _agg_kernel(n_sp, d, n_chunks):
  mesh = plsc.VectorSubcoreMesh(core_axis_name="c", subcore_axis_name="s")
  zslice = n_sp // NS

  def body(src_hbm, dst_hbm, table_hbm, out_hbm,
           src_v, dst_v, rows_v, zb_v, acc_sp, *sems):
    c = lax.axis_index("c")
    s = lax.axis_index("s")
    wid = s * NC + c
    gsem = sems[:NBUF]
    ssem = sems[NBUF:]

    def fill_z(i, _):
      zb_v[i, :] = jnp.zeros((16,), jnp.float32)
      return _
    lax.fori_loop(0, ZROWS, fill_z, None)
    for k in range(zslice // ZROWS):
      pltpu.sync_copy(zb_v, acc_sp.at[pl.ds(s * zslice + k * ZROWS, ZROWS)])
    plsc.subcore_barrier()

    def chunk(i, _):
      row0 = (wid * n_chunks + i) * GROUPS
      pltpu.sync_copy(src_hbm.at[pl.ds(row0, GROUPS)], src_v)
      pltpu.sync_copy(dst_hbm.at[pl.ds(row0, GROUPS)], dst_v)
      g = [None] * NBUF
      sc = [None] * NBUF
      pend = [False] * NBUF
      for j in range(PRIME):
        g[j % NBUF] = pltpu.async_copy(
            table_hbm.at[src_v.at[j]], rows_v.at[j % NBUF], gsem[j % NBUF])
      for j in range(GROUPS):
        b = j % NBUF
        g[b].wait()
        sc[b] = pltpu.async_copy(
            rows_v.at[b], acc_sp.at[dst_v.at[j]], ssem[b], add=True)
        pend[b] = True
        nj = j + PRIME
        if nj < GROUPS:
          nb = nj % NBUF
          if pend[nb]:
            sc[nb].wait()
            pend[nb] = False
          g[nb] = pltpu.async_copy(
              table_hbm.at[src_v.at[nj]], rows_v.at[nb], gsem[nb])
      for b in range(NBUF):
        if pend[b]:
          sc[b].wait()
      return _
    lax.fori_loop(0, n_chunks, chunk, None)

    plsc.subcore_barrier()
    pltpu.sync_copy(acc_sp.at[pl.ds(s * zslice, zslice)],
                    out_hbm.at[c, pl.ds(s * zslice, zslice)])

  return pl.kernel(
      body,
      out_type=jax.ShapeDtypeStruct((NC, n_sp, d), jnp.float32),
      mesh=mesh,
      compiler_params=pltpu.CompilerParams(use_tc_tiling_on_sc=False),
      scratch_types=[
          pltpu.VMEM((GROUPS, LANES), jnp.int32),
          pltpu.VMEM((GROUPS, LANES), jnp.int32),
          pltpu.VMEM((NBUF, LANES, d), jnp.float32),
          pltpu.VMEM((ZROWS, d), jnp.float32),
          pltpu.VMEM_SHARED((n_sp, d), jnp.float32),
      ] + [pltpu.SemaphoreType.DMA] * (2 * NBUF),
  )


_BR = 2000


def _stage_a_body(dp0, dp1, x, w1, dis_o, g1_o):
  deg = dp0[:] + dp1[:] + 1.0
  dis = lax.rsqrt(deg)
  dis_o[:] = dis
  g1_o[:] = jnp.dot(x[:], w1[:], preferred_element_type=jnp.float32) * dis


def _stage_b_body(a0, a1, g1, dis, b1, u_o):
  total = (a0[:] + a1[:] + g1[:]) * dis[:] + b1[:]
  u_o[:] = jnp.maximum(total, 0.0) * dis[:]


def _stage_c_body(a0, a1, u, dis, w2, b2, out_o):
  v = (a0[:] + a1[:] + u[:]) * dis[:]
  o = jnp.dot(v, w2[:], preferred_element_type=jnp.float32) + b2[:]
  m = jnp.max(o, axis=1, keepdims=True)
  lse = m + jnp.log(jnp.sum(jnp.exp(o - m), axis=1, keepdims=True))
  out_o[:] = o - lse


def _row_spec(d):
  return pl.BlockSpec((_BR, d), lambda i: (i, 0))


def _full_spec(r, d):
  return pl.BlockSpec((r, d), lambda i: (0, 0))


def kernel(x, edge_index, W1, b1, W2, b2):
  n, d_in = x.shape
  e = edge_index.shape[1]
  d_hid = W1.shape[1]
  d_out = W2.shape[1]
  n_sp = _node_pad(n)
  pad_row = n

  n_chunks = math.ceil(e / (NW * CHUNK))
  e_pad = NW * n_chunks * CHUNK
  src = edge_index[0]
  dst = edge_index[1]
  if e_pad > e:
    src = jnp.concatenate([src, jnp.zeros((e_pad - e,), src.dtype)])
    dst = jnp.concatenate([dst, jnp.full((e_pad - e,), pad_row, dst.dtype)])
  src2 = src.reshape(-1, LANES)
  dst2 = dst.reshape(-1, LANES)

  hist = _hist_kernel(n_sp, n_sp // NS, n_chunks)(dst2)
  dp0 = hist[0, :n].reshape(n, 1)
  dp1 = hist[1, :n].reshape(n, 1)

  grid = (n // _BR,)
  dis, g1 = pl.pallas_call(
      _stage_a_body,
      grid=grid,
      in_specs=[_row_spec(1), _row_spec(1), _row_spec(d_in),
                _full_spec(d_in, d_hid)],
      out_specs=[_row_spec(1), _row_spec(d_hid)],
      out_shape=[jax.ShapeDtypeStruct((n, 1), jnp.float32),
                 jax.ShapeDtypeStruct((n, d_hid), jnp.float32)],
  )(dp0, dp1, x, W1)

  agg = _agg_kernel(n_sp, d_hid, n_chunks)
  acc1 = agg(src2, dst2, g1)

  u = pl.pallas_call(
      _stage_b_body,
      grid=grid,
      in_specs=[_row_spec(d_hid), _row_spec(d_hid), _row_spec(d_hid),
                _row_spec(1), _full_spec(1, d_hid)],
      out_specs=_row_spec(d_hid),
      out_shape=jax.ShapeDtypeStruct((n, d_hid), jnp.float32),
  )(acc1[0, :n], acc1[1, :n], g1, dis, b1.reshape(1, d_hid))

  acc2 = agg(src2, dst2, u)

  out = pl.pallas_call(
      _stage_c_body,
      grid=grid,
      in_specs=[_row_spec(d_hid), _row_spec(d_hid), _row_spec(d_hid),
                _row_spec(1), _full_spec(d_hid, d_out),
                _full_spec(1, d_out)],
      out_specs=_row_spec(d_out),
      out_shape=jax.ShapeDtypeStruct((n, d_out), jnp.float32),
  )(acc2[0, :n], acc2[1, :n], u, dis, W2, b2.reshape(1, d_out))

  return out
